# baseline (speedup 1.0000x reference)
import jax
import jax.numpy as jnp
from jax import lax
from jax.experimental import pallas as pl
from jax.experimental.pallas import tpu as pltpu
from jax.experimental.pallas import tpu_sc as plsc

BATCH = 16384
EMBED_DIM = 32
NUM_TABLES = 4
OUT_W = NUM_TABLES * EMBED_DIM

_info = plsc.get_sparse_core_info()
_NC, _NS = _info.num_cores, _info.num_subcores
_NW = _NC * _NS
_BPW = BATCH // _NW


def _gather_concat_kernel(rid_hbm, reg_hbm, ovr_hbm, asin_hbm,
                          u_hbm, r_hbm, o_hbm, p_hbm, out_hbm,
                          idx_v, comb_v, r_sh, o_sh, p_sh, sem, asem, wsem):
    sid = lax.axis_index("s")
    wid = sid * _NC + lax.axis_index("c")
    base = wid * _BPW
    idx_hbms = (rid_hbm, reg_hbm, ovr_hbm, asin_hbm)
    icopies = [pltpu.async_copy(idx_hbms[t].at[pl.ds(base, _BPW)],
                                idx_v.at[t], sem)
               for t in range(NUM_TABLES)]
    icopies[0].wait()
    ucopy = pltpu.async_copy(u_hbm.at[idx_v.at[0]], comb_v, sem)
    @pl.when(sid == 0)
    def _stage():
        pltpu.sync_copy(r_hbm, r_sh)
        pltpu.sync_copy(o_hbm, o_sh)
        pltpu.sync_copy(p_hbm, p_sh)
    plsc.subcore_barrier()
    for t in range(1, NUM_TABLES):
        icopies[t].wait()
    ucopy.wait()
    tables_sh = (r_sh, o_sh, p_sh)
    acopies = [pltpu.async_copy(tables_sh[t - 1].at[idx_v.at[t]], comb_v,
                                asem, add=True)
               for t in range(1, NUM_TABLES)]
    for c in acopies:
        c.wait()
    pltpu.async_copy(comb_v, out_hbm.at[pl.ds(base, _BPW)], wsem).wait()


@jax.jit
def _run(rid, reg, ovr, asin, u128, r128, o128, p128):
    mesh = plsc.VectorSubcoreMesh(core_axis_name="c", subcore_axis_name="s")
    return pl.kernel(
        _gather_concat_kernel,
        out_type=jax.ShapeDtypeStruct((BATCH, OUT_W), jnp.float32),
        mesh=mesh,
        scratch_types=[
            pltpu.VMEM((NUM_TABLES, _BPW), jnp.int32),
            pltpu.VMEM((_BPW, OUT_W), jnp.float32),
            pltpu.VMEM_SHARED(r128.shape, jnp.float32),
            pltpu.VMEM_SHARED(o128.shape, jnp.float32),
            pltpu.VMEM_SHARED(p128.shape, jnp.float32),
            pltpu.SemaphoreType.DMA,
            pltpu.SemaphoreType.DMA,
            pltpu.SemaphoreType.DMA,
        ],
        compiler_params=pltpu.CompilerParams(use_tc_tiling_on_sc=False,
                                             needs_layout_passes=False),
    )(rid, reg, ovr, asin, u128, r128, o128, p128)


def kernel(reviewerID, region, overall, asin, user_table, region_table,
           rating_table, product_table):
    u128 = jnp.pad(user_table, ((0, 0), (0, 3 * EMBED_DIM)))
    r128 = jnp.pad(region_table, ((0, 0), (EMBED_DIM, 2 * EMBED_DIM)))
    o128 = jnp.pad(rating_table, ((0, 0), (2 * EMBED_DIM, EMBED_DIM)))
    p128 = jnp.pad(product_table, ((0, 0), (3 * EMBED_DIM, 0)))
    return _run(reviewerID.astype(jnp.int32), region.astype(jnp.int32),
                overall.astype(jnp.int32), asin.astype(jnp.int32),
                u128, r128, o128, p128)


# speedup vs baseline: 1.1901x; 1.1901x over previous
import jax
import jax.numpy as jnp
from jax import lax
from jax.experimental import pallas as pl
from jax.experimental.pallas import tpu as pltpu
from jax.experimental.pallas import tpu_sc as plsc

BATCH = 16384
EMBED_DIM = 32
NUM_TABLES = 4
NUM_USERS1 = 100001

_info = plsc.get_sparse_core_info()
_NC, _NS = _info.num_cores, _info.num_subcores
_NW = _NC * _NS
_BPW = BATCH // _NW
_L = 16
_NG = _BPW // _L


def _gather_concat_kernel(rid_hbm, reg_hbm, ovr_hbm, asin_hbm,
                          uflat_hbm, r_hbm, o_hbm, p_hbm, out_hbm,
                          idx_v, eidx_v, urows_v, rows_v, r_sh, o_sh, p_sh,
                          isem, usem, ssem, wsem):
    sid = lax.axis_index("s")
    wid = sid * _NC + lax.axis_index("c")
    base = wid * _BPW
    idx_hbms = (rid_hbm, reg_hbm, ovr_hbm, asin_hbm)
    icopies = [pltpu.async_copy(idx_hbms[t].at[pl.ds(base, _BPW)],
                                idx_v.at[t], isem)
               for t in range(NUM_TABLES)]
    icopies[0].wait()

    lanes = lax.iota(jnp.int32, _L)

    def group_body(g, carry):
        uvec = idx_v[0, pl.ds(g * _L, _L)]
        gvec = jnp.full((_L,), g, jnp.int32)
        dvec = lanes * EMBED_DIM
        for c in range(EMBED_DIM):
            plsc.store_scatter(eidx_v, [gvec, dvec + c],
                               uvec + c * NUM_USERS1)
        return carry

    lax.fori_loop(0, _NG, group_body, 0)
    ucopies = [
        pltpu.async_copy(uflat_hbm.at[eidx_v.at[g]],
                         urows_v.at[pl.ds(g * _BPW, _BPW)], usem)
        for g in range(_NG)
    ]
    @pl.when(sid == 0)
    def _stage():
        pltpu.sync_copy(r_hbm, r_sh)
        pltpu.sync_copy(o_hbm, o_sh)
        pltpu.sync_copy(p_hbm, p_sh)
    plsc.subcore_barrier()
    tables_sh = (r_sh, o_sh, p_sh)
    copies = []
    for t in range(1, NUM_TABLES):
        icopies[t].wait()
        copies.append(pltpu.async_copy(
            tables_sh[t - 1].at[idx_v.at[t]], rows_v.at[t], ssem))
    for c in ucopies:
        c.wait()

    def repack_body(q, carry):
        b0 = q * 4
        for j in range(4):
            b = b0 + j
            rows_v[0, b, pl.ds(0, _L)] = urows_v[pl.ds(b * EMBED_DIM, _L)]
            rows_v[0, b, pl.ds(_L, _L)] = (
                urows_v[pl.ds(b * EMBED_DIM + _L, _L)])
        return carry

    lax.fori_loop(0, _BPW // 4, repack_body, 0)
    for c in copies:
        c.wait()
    wcopies = []
    for t in range(NUM_TABLES):
        wcopies.append(pltpu.async_copy(
            rows_v.at[t],
            out_hbm.at[pl.ds(base, _BPW), pl.ds(t * EMBED_DIM, EMBED_DIM)],
            wsem))
    for c in wcopies:
        c.wait()


@jax.jit
def _run(rid, reg, ovr, asin, uflat, region_table, rating_table,
         product_table):
    mesh = plsc.VectorSubcoreMesh(core_axis_name="c", subcore_axis_name="s")
    return pl.kernel(
        _gather_concat_kernel,
        out_type=jax.ShapeDtypeStruct((BATCH, NUM_TABLES * EMBED_DIM),
                                      jnp.float32),
        mesh=mesh,
        scratch_types=[
            pltpu.VMEM((NUM_TABLES, _BPW), jnp.int32),
            pltpu.VMEM((_NG, _BPW), jnp.int32),
            pltpu.VMEM((_BPW * EMBED_DIM,), jnp.float32),
            pltpu.VMEM((NUM_TABLES, _BPW, EMBED_DIM), jnp.float32),
            pltpu.VMEM_SHARED(region_table.shape, jnp.float32),
            pltpu.VMEM_SHARED(rating_table.shape, jnp.float32),
            pltpu.VMEM_SHARED(product_table.shape, jnp.float32),
            pltpu.SemaphoreType.DMA,
            pltpu.SemaphoreType.DMA,
            pltpu.SemaphoreType.DMA,
            pltpu.SemaphoreType.DMA,
        ],
        compiler_params=pltpu.CompilerParams(use_tc_tiling_on_sc=False,
                                             needs_layout_passes=False),
    )(rid, reg, ovr, asin, uflat, region_table, rating_table,
      product_table)


def kernel(reviewerID, region, overall, asin, user_table, region_table,
           rating_table, product_table):
    uflat = jnp.reshape(user_table.T, (-1,))
    return _run(reviewerID.astype(jnp.int32), region.astype(jnp.int32),
                overall.astype(jnp.int32), asin.astype(jnp.int32),
                uflat, region_table, rating_table, product_table)


# speedup vs baseline: 1.3309x; 1.1183x over previous
import jax
import jax.numpy as jnp
from jax import lax
from jax.experimental import pallas as pl
from jax.experimental.pallas import tpu as pltpu
from jax.experimental.pallas import tpu_sc as plsc

BATCH = 16384
EMBED_DIM = 32
NUM_TABLES = 4
NUM_USERS1 = 100001

_info = plsc.get_sparse_core_info()
_NC, _NS = _info.num_cores, _info.num_subcores
_NW = _NC * _NS
_BPW = BATCH // _NW
_L = 16
_NG = _BPW // _L


def _gather_concat_kernel(rid_hbm, reg_hbm, ovr_hbm, asin_hbm,
                          uflat_hbm, r_hbm, o_hbm, p_hbm, out_hbm,
                          idx_v, eidx_v, urows_v, rows_v, r_sh, o_sh, p_sh,
                          isem, usem, ssem, wsem):
    sid = lax.axis_index("s")
    wid = sid * _NC + lax.axis_index("c")
    base = wid * _BPW
    idx_hbms = (rid_hbm, reg_hbm, ovr_hbm, asin_hbm)
    icopies = [pltpu.async_copy(idx_hbms[t].at[pl.ds(base, _BPW)],
                                idx_v.at[t], isem)
               for t in range(NUM_TABLES)]
    icopies[0].wait()

    lanes = lax.iota(jnp.int32, _L)

    def group_body(g, carry):
        uvec = idx_v[0, pl.ds(g * _L, _L)]
        gvec = jnp.full((_L,), g, jnp.int32)
        dvec = lanes * EMBED_DIM
        for c in range(EMBED_DIM):
            plsc.store_scatter(eidx_v, [gvec, dvec + c],
                               uvec + c * NUM_USERS1)
        pltpu.async_copy(uflat_hbm.at[eidx_v.at[g]],
                         urows_v.at[pl.ds(g * _BPW, _BPW)], usem)
        return carry

    lax.fori_loop(0, _NG, group_body, 0)
    udrain = pltpu.make_async_copy(uflat_hbm.at[pl.ds(0, _BPW * EMBED_DIM)],
                                   urows_v, usem)
    @pl.when(sid == 0)
    def _stage():
        pltpu.sync_copy(r_hbm, r_sh)
        pltpu.sync_copy(o_hbm, o_sh)
        pltpu.sync_copy(p_hbm, p_sh)
    plsc.subcore_barrier()
    tables_sh = (r_sh, o_sh, p_sh)
    copies = []
    for t in range(1, NUM_TABLES):
        icopies[t].wait()
        copies.append(pltpu.async_copy(
            tables_sh[t - 1].at[idx_v.at[t]], rows_v.at[t], ssem))
    udrain.wait()

    def repack_body(q, carry):
        b0 = q * 4
        for j in range(4):
            b = b0 + j
            rows_v[0, b, pl.ds(0, _L)] = urows_v[pl.ds(b * EMBED_DIM, _L)]
            rows_v[0, b, pl.ds(_L, _L)] = (
                urows_v[pl.ds(b * EMBED_DIM + _L, _L)])
        return carry

    lax.fori_loop(0, _BPW // 4, repack_body, 0)
    for c in copies:
        c.wait()
    wcopies = []
    for t in range(NUM_TABLES):
        wcopies.append(pltpu.async_copy(
            rows_v.at[t],
            out_hbm.at[pl.ds(base, _BPW), pl.ds(t * EMBED_DIM, EMBED_DIM)],
            wsem))
    for c in wcopies:
        c.wait()


@jax.jit
def _run(rid, reg, ovr, asin, uflat, region_table, rating_table,
         product_table):
    mesh = plsc.VectorSubcoreMesh(core_axis_name="c", subcore_axis_name="s")
    return pl.kernel(
        _gather_concat_kernel,
        out_type=jax.ShapeDtypeStruct((BATCH, NUM_TABLES * EMBED_DIM),
                                      jnp.float32),
        mesh=mesh,
        scratch_types=[
            pltpu.VMEM((NUM_TABLES, _BPW), jnp.int32),
            pltpu.VMEM((_NG, _BPW), jnp.int32),
            pltpu.VMEM((_BPW * EMBED_DIM,), jnp.float32),
            pltpu.VMEM((NUM_TABLES, _BPW, EMBED_DIM), jnp.float32),
            pltpu.VMEM_SHARED(region_table.shape, jnp.float32),
            pltpu.VMEM_SHARED(rating_table.shape, jnp.float32),
            pltpu.VMEM_SHARED(product_table.shape, jnp.float32),
            pltpu.SemaphoreType.DMA,
            pltpu.SemaphoreType.DMA,
            pltpu.SemaphoreType.DMA,
            pltpu.SemaphoreType.DMA,
        ],
        compiler_params=pltpu.CompilerParams(use_tc_tiling_on_sc=False,
                                             needs_layout_passes=False),
    )(rid, reg, ovr, asin, uflat, region_table, rating_table,
      product_table)


def kernel(reviewerID, region, overall, asin, user_table, region_table,
           rating_table, product_table):
    uflat = jnp.reshape(user_table.T, (-1,))
    return _run(reviewerID.astype(jnp.int32), region.astype(jnp.int32),
                overall.astype(jnp.int32), asin.astype(jnp.int32),
                uflat, region_table, rating_table, product_table)
